# Initial kernel scaffold; baseline (speedup 1.0000x reference)
import jax
import jax.numpy as jnp
from jax.experimental import pallas as pl


def kernel(conv1_w, conv1_b, conv2_w, conv2_b, conv3_w, conv3_b, fc_w, fc_b, embed_fc_w, embed_fc_b, img, embed):
    raise NotImplementedError("write your pallas kernel here")


# speedup vs baseline: 172.1559x; 172.1559x over previous
import jax
import jax.numpy as jnp
from jax.experimental import pallas as pl
from jax.experimental.pallas import tpu as pltpu


def _deinterleave(x_i32):
    lo = jax.lax.bitcast_convert_type(x_i32.astype(jnp.int16), jnp.bfloat16)
    hi = jax.lax.bitcast_convert_type(
        jax.lax.shift_right_logical(x_i32, 16).astype(jnp.int16),
        jnp.bfloat16)
    return jnp.concatenate([lo, hi], axis=-1)


def _cols_k4s2(y, Ho):
    Bt, H, _, C = y.shape
    G = H // 2 + 1
    yw = jnp.pad(y, ((0, 0), (0, 0), (1, 1), (0, 0)))
    yw = _deinterleave(pltpu.bitcast(yw, jnp.int32))
    cw = jnp.concatenate([yw[:, :, 0:Ho, :], yw[:, :, 1:Ho + 1, :]],
                         axis=-1)
    hp = jnp.pad(cw, ((0, 0), (1, 1), (0, 0), (0, 0)))
    hp = hp.reshape(Bt, G, 2, Ho, 4 * C)
    ev, od = hp[:, :, 0], hp[:, :, 1]
    cols = jnp.concatenate(
        [ev[:, 0:Ho], od[:, 0:Ho], ev[:, 1:Ho + 1], od[:, 1:Ho + 1]],
        axis=-1)
    return cols.reshape(Bt * Ho * Ho, 16 * C)


def _fwd_kernel(c1_ref, w1_ref, b1_ref, w2_ref, b2_ref, w3_ref, b3_ref,
                fcw_ref, fcb_ref, o_ref):
    Bt = c1_ref.shape[0] // 256

    y1 = jnp.dot(c1_ref[...], w1_ref[...],
                 preferred_element_type=jnp.float32)
    y1 = y1 + b1_ref[...]
    y1 = jnp.where(y1 > 0, y1, 0.2 * y1).astype(jnp.bfloat16)

    c2 = _cols_k4s2(y1.reshape(Bt, 16, 16, 64), 8)
    y2 = jnp.dot(c2, w2_ref[...], preferred_element_type=jnp.float32)
    y2 = y2 + b2_ref[...]
    y2 = jnp.where(y2 > 0, y2, 0.2 * y2).astype(jnp.bfloat16)

    c3 = _cols_k4s2(y2.reshape(Bt, 8, 8, 128), 4)
    y3 = jnp.dot(c3, w3_ref[...], preferred_element_type=jnp.float32)
    y3 = y3 + b3_ref[...]
    y3 = jnp.where(y3 > 0, y3, 0.2 * y3)

    prod = y3.reshape(Bt, 16, 256) * fcw_ref[...]
    s = jnp.sum(prod, axis=(1, 2), keepdims=True).reshape(Bt, 1)
    o_ref[...] = jax.nn.sigmoid(s + fcb_ref[...])


def kernel(conv1_w, conv1_b, conv2_w, conv2_b, conv3_w, conv3_b,
           fc_w, fc_b, embed_fc_w, embed_fc_b, img, embed):
    B = img.shape[0]

    ef = embed @ embed_fc_w + embed_fc_b
    ef_img = jnp.broadcast_to(ef.reshape(B, 4, 1, 4, 1),
                              (B, 4, 8, 4, 8)).reshape(B, 1, 32, 32)
    x = jnp.concatenate([img, ef_img], axis=1)
    x = jnp.transpose(x, (0, 2, 3, 1)).astype(jnp.bfloat16)
    xp = jnp.pad(x, ((0, 0), (1, 1), (1, 1), (0, 0)))
    taps = [xp[:, kh:kh + 31:2, kw:kw + 31:2, :]
            for kh in range(4) for kw in range(4)]
    c1 = jnp.concatenate(taps, axis=-1).reshape(B * 256, 64)

    fcw = fc_w.astype(jnp.float32).reshape(1, 16, 256)
    fcb = fc_b.astype(jnp.float32).reshape(1, 1)

    Bt = B
    for cand in (64, 32, 16, 8, 4, 2, 1):
        if B % cand == 0:
            Bt = cand
            break

    out = pl.pallas_call(
        _fwd_kernel,
        out_shape=jax.ShapeDtypeStruct((B, 1), jnp.float32),
        grid=(B // Bt,),
        in_specs=[
            pl.BlockSpec((Bt * 256, 64), lambda i: (i, 0)),
            pl.BlockSpec((64, 64), lambda i: (0, 0)),
            pl.BlockSpec((1, 64), lambda i: (0, 0)),
            pl.BlockSpec((1024, 128), lambda i: (0, 0)),
            pl.BlockSpec((1, 128), lambda i: (0, 0)),
            pl.BlockSpec((2048, 256), lambda i: (0, 0)),
            pl.BlockSpec((1, 256), lambda i: (0, 0)),
            pl.BlockSpec((1, 16, 256), lambda i: (0, 0, 0)),
            pl.BlockSpec((1, 1), lambda i: (0, 0)),
        ],
        out_specs=pl.BlockSpec((Bt, 1), lambda i: (i, 0)),
        compiler_params=pltpu.CompilerParams(
            dimension_semantics=("parallel",)),
        cost_estimate=pl.CostEstimate(
            flops=2 * B * (256 * 64 * 64 + 64 * 1024 * 128 + 16 * 2048 * 256),
            transcendentals=B,
            bytes_accessed=B * 256 * 64 * 2 + B * 4),
    )(c1, conv1_w, conv1_b, conv2_w, conv2_b, conv3_w, conv3_b, fcw, fcb)
    return out


# speedup vs baseline: 374.1078x; 2.1731x over previous
import jax
import jax.numpy as jnp
from jax.experimental import pallas as pl
from jax.experimental.pallas import tpu as pltpu


def _deinterleave(x_i32):
    lo = jax.lax.bitcast_convert_type(x_i32.astype(jnp.int16), jnp.bfloat16)
    hi = jax.lax.bitcast_convert_type(
        jax.lax.shift_right_logical(x_i32, 16).astype(jnp.int16),
        jnp.bfloat16)
    return jnp.concatenate([lo, hi], axis=-1)


def _cols_conv1(xt):
    Bt = xt.shape[0]
    z = _deinterleave(pltpu.bitcast(xt, jnp.int32))
    ev, od = z[:, :, 0:136], z[:, :, 136:272]
    p = [ev[:, 0:16], od[:, 0:16], ev[:, 1:17], od[:, 1:17]]
    rows = []
    for i in range(16):
        rows.append(jnp.concatenate(
            [t[:, :, 8 * i:8 * i + 16] for t in p], axis=-1))
    return jnp.concatenate(rows, axis=0).reshape(16 * Bt * 16, 64)


def _cols_k4s2(y, Ho):
    H, Bt, _, C = y.shape
    G = H // 2 + 1
    yw = jnp.pad(y, ((0, 0), (0, 0), (1, 1), (0, 0)))
    yw = _deinterleave(pltpu.bitcast(yw, jnp.int32))
    cw = jnp.concatenate([yw[:, :, 0:Ho, :], yw[:, :, 1:Ho + 1, :]],
                         axis=-1)
    hp = jnp.pad(cw, ((1, 1), (0, 0), (0, 0), (0, 0)))
    hp = hp.reshape(G, 2, Bt, Ho, 4 * C)
    ev, od = hp[:, 0], hp[:, 1]
    cols = jnp.concatenate(
        [ev[0:Ho], od[0:Ho], ev[1:Ho + 1], od[1:Ho + 1]],
        axis=-1)
    return cols.reshape(Ho * Bt * Ho, 16 * C)


def _fwd_kernel(xt_ref, w1_ref, b1_ref, w2_ref, b2_ref, w3_ref, b3_ref,
                fcw_ref, fcb_ref, o_ref):
    Bt = xt_ref.shape[0]

    c1 = _cols_conv1(xt_ref[...])
    y1 = jnp.dot(c1, w1_ref[...], preferred_element_type=jnp.float32)
    y1 = y1 + b1_ref[...]
    y1 = jnp.where(y1 > 0, y1, 0.2 * y1).astype(jnp.bfloat16)

    c2 = _cols_k4s2(y1.reshape(16, Bt, 16, 64), 8)
    y2 = jnp.dot(c2, w2_ref[...], preferred_element_type=jnp.float32)
    y2 = y2 + b2_ref[...]
    y2 = jnp.where(y2 > 0, y2, 0.2 * y2).astype(jnp.bfloat16)

    c3 = _cols_k4s2(y2.reshape(8, Bt, 8, 128), 4)
    y3 = jnp.dot(c3, w3_ref[...], preferred_element_type=jnp.float32)
    y3 = y3 + b3_ref[...]
    y3 = jnp.where(y3 > 0, y3, 0.2 * y3)

    prod = y3.reshape(4, Bt, 4, 256) * fcw_ref[...]
    s = jnp.sum(prod, axis=0)
    s = jnp.sum(s, axis=(1, 2), keepdims=True).reshape(Bt, 1)
    o_ref[...] = jax.nn.sigmoid(s + fcb_ref[...])


def kernel(conv1_w, conv1_b, conv2_w, conv2_b, conv3_w, conv3_b,
           fc_w, fc_b, embed_fc_w, embed_fc_b, img, embed):
    B = img.shape[0]

    ef = embed @ embed_fc_w + embed_fc_b
    ef_img = jnp.broadcast_to(ef.reshape(B, 4, 1, 4, 1),
                              (B, 4, 8, 4, 8)).reshape(B, 1, 32, 32)
    x = jnp.concatenate([img, ef_img], axis=1)
    x = jnp.transpose(x, (0, 3, 2, 1)).astype(jnp.bfloat16)
    xt = jnp.pad(x, ((0, 0), (1, 1), (1, 1), (0, 0))).reshape(B, 34, 136)

    w1 = conv1_w.reshape(4, 4, 4, 64).transpose(1, 0, 2, 3).reshape(64, 64)
    fcw = fc_w.astype(jnp.float32).reshape(4, 1, 4, 256)
    fcb = fc_b.astype(jnp.float32).reshape(1, 1)

    Bt = B
    for cand in (64, 32, 16, 8, 4, 2, 1):
        if B % cand == 0:
            Bt = cand
            break

    out = pl.pallas_call(
        _fwd_kernel,
        out_shape=jax.ShapeDtypeStruct((B, 1), jnp.float32),
        grid=(B // Bt,),
        in_specs=[
            pl.BlockSpec((Bt, 34, 136), lambda i: (i, 0, 0)),
            pl.BlockSpec((64, 64), lambda i: (0, 0)),
            pl.BlockSpec((1, 64), lambda i: (0, 0)),
            pl.BlockSpec((1024, 128), lambda i: (0, 0)),
            pl.BlockSpec((1, 128), lambda i: (0, 0)),
            pl.BlockSpec((2048, 256), lambda i: (0, 0)),
            pl.BlockSpec((1, 256), lambda i: (0, 0)),
            pl.BlockSpec((4, 1, 4, 256), lambda i: (0, 0, 0, 0)),
            pl.BlockSpec((1, 1), lambda i: (0, 0)),
        ],
        out_specs=pl.BlockSpec((Bt, 1), lambda i: (i, 0)),
        compiler_params=pltpu.CompilerParams(
            dimension_semantics=("parallel",)),
        cost_estimate=pl.CostEstimate(
            flops=2 * B * (256 * 64 * 64 + 64 * 1024 * 128 + 16 * 2048 * 256),
            transcendentals=B,
            bytes_accessed=B * 34 * 136 * 2 + B * 4),
    )(xt, w1, conv1_b, conv2_w, conv2_b, conv3_w, conv3_b, fcw, fcb)
    return out


# speedup vs baseline: 374.2187x; 1.0003x over previous
import jax
import jax.numpy as jnp
from jax.experimental import pallas as pl
from jax.experimental.pallas import tpu as pltpu


def _deinterleave(x_i32):
    lo = jax.lax.bitcast_convert_type(x_i32.astype(jnp.int16), jnp.bfloat16)
    hi = jax.lax.bitcast_convert_type(
        jax.lax.shift_right_logical(x_i32, 16).astype(jnp.int16),
        jnp.bfloat16)
    return jnp.concatenate([lo, hi], axis=-1)


def _cols_conv1(xt):
    Bt = xt.shape[0]
    z = _deinterleave(pltpu.bitcast(xt, jnp.int32))
    ev, od = z[:, :, 0:136], z[:, :, 136:272]
    p = [ev[:, 0:16], od[:, 0:16], ev[:, 1:17], od[:, 1:17]]
    rows = []
    for i in range(16):
        rows.append(jnp.concatenate(
            [t[:, :, 8 * i:8 * i + 16] for t in p], axis=-1))
    return jnp.concatenate(rows, axis=0).reshape(16 * Bt * 16, 64)


def _cols_k4s2(y, Ho):
    H, Bt, _, C = y.shape
    G = H // 2 + 1
    yw = jnp.pad(y, ((0, 0), (0, 0), (1, 1), (0, 0)))
    yw = _deinterleave(pltpu.bitcast(yw, jnp.int32))
    cw = jnp.concatenate([yw[:, :, 0:Ho, :], yw[:, :, 1:Ho + 1, :]],
                         axis=-1)
    hp = jnp.pad(cw, ((1, 1), (0, 0), (0, 0), (0, 0)))
    hp = hp.reshape(G, 2, Bt, Ho, 4 * C)
    ev, od = hp[:, 0], hp[:, 1]
    cols = jnp.concatenate(
        [ev[0:Ho], od[0:Ho], ev[1:Ho + 1], od[1:Ho + 1]],
        axis=-1)
    return cols.reshape(Ho * Bt * Ho, 16 * C)


def _fwd_kernel(xt_ref, w1_ref, b1_ref, w2_ref, b2_ref, w3_ref, b3_ref,
                fcw_ref, fcb_ref, o_ref):
    Bt = xt_ref.shape[0]

    c1 = _cols_conv1(xt_ref[...])
    y1 = jnp.dot(c1, w1_ref[...], preferred_element_type=jnp.float32)
    y1 = y1 + b1_ref[...]
    y1 = jnp.where(y1 > 0, y1, 0.2 * y1).astype(jnp.bfloat16)

    c2 = _cols_k4s2(y1.reshape(16, Bt, 16, 64), 8)
    y2 = jnp.dot(c2, w2_ref[...], preferred_element_type=jnp.float32)
    y2 = y2 + b2_ref[...]
    y2 = jnp.where(y2 > 0, y2, 0.2 * y2).astype(jnp.bfloat16)

    c3 = _cols_k4s2(y2.reshape(8, Bt, 8, 128), 4)
    y3 = jnp.dot(c3, w3_ref[...], preferred_element_type=jnp.float32)
    y3 = y3 + b3_ref[...]
    y3 = jnp.where(y3 > 0, y3, 0.2 * y3)

    prod = y3.reshape(4, Bt, 4, 256) * fcw_ref[...]
    s = jnp.sum(prod, axis=0)
    s = jnp.sum(s, axis=(1, 2), keepdims=True).reshape(Bt, 1)
    o_ref[...] = jax.nn.sigmoid(s + fcb_ref[...])


def kernel(conv1_w, conv1_b, conv2_w, conv2_b, conv3_w, conv3_b,
           fc_w, fc_b, embed_fc_w, embed_fc_b, img, embed):
    B = img.shape[0]

    ef = (embed @ embed_fc_w + embed_fc_b).astype(jnp.bfloat16)
    ef_img = jnp.broadcast_to(
        jnp.transpose(ef.reshape(B, 4, 4), (0, 2, 1)).reshape(B, 4, 1, 4, 1),
        (B, 4, 8, 4, 8)).reshape(B, 32, 32, 1)
    imgt = jnp.transpose(img, (0, 3, 2, 1)).astype(jnp.bfloat16)
    x = jnp.concatenate([imgt, ef_img], axis=-1)
    xt = jnp.pad(x, ((0, 0), (1, 1), (1, 1), (0, 0))).reshape(B, 34, 136)

    w1 = conv1_w.reshape(4, 4, 4, 64).transpose(1, 0, 2, 3).reshape(64, 64)
    fcw = fc_w.astype(jnp.float32).reshape(4, 1, 4, 256)
    fcb = fc_b.astype(jnp.float32).reshape(1, 1)

    Bt = B
    for cand in (64, 32, 16, 8, 4, 2, 1):
        if B % cand == 0:
            Bt = cand
            break

    out = pl.pallas_call(
        _fwd_kernel,
        out_shape=jax.ShapeDtypeStruct((B, 1), jnp.float32),
        grid=(B // Bt,),
        in_specs=[
            pl.BlockSpec((Bt, 34, 136), lambda i: (i, 0, 0)),
            pl.BlockSpec((64, 64), lambda i: (0, 0)),
            pl.BlockSpec((1, 64), lambda i: (0, 0)),
            pl.BlockSpec((1024, 128), lambda i: (0, 0)),
            pl.BlockSpec((1, 128), lambda i: (0, 0)),
            pl.BlockSpec((2048, 256), lambda i: (0, 0)),
            pl.BlockSpec((1, 256), lambda i: (0, 0)),
            pl.BlockSpec((4, 1, 4, 256), lambda i: (0, 0, 0, 0)),
            pl.BlockSpec((1, 1), lambda i: (0, 0)),
        ],
        out_specs=pl.BlockSpec((Bt, 1), lambda i: (i, 0)),
        compiler_params=pltpu.CompilerParams(
            dimension_semantics=("parallel",)),
        cost_estimate=pl.CostEstimate(
            flops=2 * B * (256 * 64 * 64 + 64 * 1024 * 128 + 16 * 2048 * 256),
            transcendentals=B,
            bytes_accessed=B * 34 * 136 * 2 + B * 4),
    )(xt, w1, conv1_b, conv2_w, conv2_b, conv3_w, conv3_b, fcw, fcb)
    return out


# speedup vs baseline: 416.2062x; 1.1122x over previous
import jax
import jax.numpy as jnp
from jax.experimental import pallas as pl
from jax.experimental.pallas import tpu as pltpu


def _deinterleave(x_i32):
    lo = jax.lax.bitcast_convert_type(x_i32.astype(jnp.int16), jnp.bfloat16)
    hi = jax.lax.bitcast_convert_type(
        jax.lax.shift_right_logical(x_i32, 16).astype(jnp.int16),
        jnp.bfloat16)
    return jnp.concatenate([lo, hi], axis=-1)


def _cols_conv1(xt):
    Bt = xt.shape[0]
    z = _deinterleave(pltpu.bitcast(xt, jnp.int32))
    ev, od = z[:, :, 0:136], z[:, :, 136:272]
    p = [ev[:, 0:16], od[:, 0:16], ev[:, 1:17], od[:, 1:17]]
    rows = []
    for i in range(16):
        rows.append(jnp.concatenate(
            [t[:, :, 8 * i:8 * i + 16] for t in p], axis=-1))
    return jnp.concatenate(rows, axis=0).reshape(16 * Bt * 16, 64)


def _cols_k4s2(y, Ho):
    H, Bt, _, C = y.shape
    G = H // 2 + 1
    yw = jnp.pad(y, ((0, 0), (0, 0), (1, 1), (0, 0)))
    yw = _deinterleave(pltpu.bitcast(yw, jnp.int32))
    cw = jnp.concatenate([yw[:, :, 0:Ho, :], yw[:, :, 1:Ho + 1, :]],
                         axis=-1)
    hp = jnp.pad(cw, ((1, 1), (0, 0), (0, 0), (0, 0)))
    hp = hp.reshape(G, 2, Bt, Ho, 4 * C)
    ev, od = hp[:, 0], hp[:, 1]
    cols = jnp.concatenate(
        [ev[0:Ho], od[0:Ho], ev[1:Ho + 1], od[1:Ho + 1]],
        axis=-1)
    return cols.reshape(Ho * Bt * Ho, 16 * C)


def _fwd_kernel(xin_ref, p_ref, w1_ref, b1_ref, w2_ref, b2_ref, w3_ref,
                b3_ref, fcw_ref, fcb_ref, o_ref):
    Bt = xin_ref.shape[0]

    xt = jnp.dot(xin_ref[...].reshape(Bt * 34, 136), p_ref[...],
                 preferred_element_type=jnp.float32)
    xt = xt.astype(jnp.bfloat16).reshape(Bt, 34, 136)

    c1 = _cols_conv1(xt)
    y1 = jnp.dot(c1, w1_ref[...], preferred_element_type=jnp.float32)
    y1 = y1 + b1_ref[...]
    y1 = jnp.where(y1 > 0, y1, 0.2 * y1).astype(jnp.bfloat16)

    c2 = _cols_k4s2(y1.reshape(16, Bt, 16, 64), 8)
    y2 = jnp.dot(c2, w2_ref[...], preferred_element_type=jnp.float32)
    y2 = y2 + b2_ref[...]
    y2 = jnp.where(y2 > 0, y2, 0.2 * y2).astype(jnp.bfloat16)

    c3 = _cols_k4s2(y2.reshape(8, Bt, 8, 128), 4)
    y3 = jnp.dot(c3, w3_ref[...], preferred_element_type=jnp.float32)
    y3 = y3 + b3_ref[...]
    y3 = jnp.where(y3 > 0, y3, 0.2 * y3)

    prod = y3.reshape(4, Bt, 4, 256) * fcw_ref[...]
    s = jnp.sum(prod, axis=0)
    s = jnp.sum(s, axis=(1, 2), keepdims=True).reshape(Bt, 1)
    o_ref[...] = jax.nn.sigmoid(s + fcb_ref[...])


def kernel(conv1_w, conv1_b, conv2_w, conv2_b, conv3_w, conv3_b,
           fc_w, fc_b, embed_fc_w, embed_fc_b, img, embed):
    B = img.shape[0]

    ef = (embed @ embed_fc_w + embed_fc_b).astype(jnp.bfloat16)
    efp = jnp.broadcast_to(
        jnp.transpose(ef.reshape(B, 4, 4), (0, 2, 1)).reshape(B, 4, 1, 4, 1),
        (B, 4, 8, 4, 8)).reshape(B, 32, 32)
    efp = jnp.pad(efp, ((0, 0), (1, 1), (1, 1)))
    imgt = jnp.transpose(img, (0, 3, 1, 2)).astype(jnp.bfloat16)
    imgt = jnp.pad(imgt, ((0, 0), (0, 0), (0, 0), (1, 1)))
    imgt = jnp.pad(imgt.reshape(B, 32, 102), ((0, 0), (1, 1), (0, 0)))
    xin = jnp.concatenate([imgt, efp], axis=-1)

    hh = jnp.arange(34)
    rows3 = (jnp.arange(3)[:, None] * 34 + hh[None, :]).reshape(-1)
    cols3 = (4 * hh[None, :] + jnp.arange(3)[:, None]).reshape(-1)
    perm = jnp.zeros((136, 136), jnp.float32)
    perm = perm.at[rows3, cols3].set(1.0)
    perm = perm.at[102 + hh, 4 * hh + 3].set(1.0).astype(jnp.bfloat16)

    w1 = conv1_w.reshape(4, 4, 4, 64).transpose(1, 0, 2, 3).reshape(64, 64)
    fcw = fc_w.astype(jnp.float32).reshape(4, 1, 4, 256)
    fcb = fc_b.astype(jnp.float32).reshape(1, 1)

    Bt = B
    for cand in (64, 32, 16, 8, 4, 2, 1):
        if B % cand == 0:
            Bt = cand
            break

    out = pl.pallas_call(
        _fwd_kernel,
        out_shape=jax.ShapeDtypeStruct((B, 1), jnp.float32),
        grid=(B // Bt,),
        in_specs=[
            pl.BlockSpec((Bt, 34, 136), lambda i: (i, 0, 0)),
            pl.BlockSpec((136, 136), lambda i: (0, 0)),
            pl.BlockSpec((64, 64), lambda i: (0, 0)),
            pl.BlockSpec((1, 64), lambda i: (0, 0)),
            pl.BlockSpec((1024, 128), lambda i: (0, 0)),
            pl.BlockSpec((1, 128), lambda i: (0, 0)),
            pl.BlockSpec((2048, 256), lambda i: (0, 0)),
            pl.BlockSpec((1, 256), lambda i: (0, 0)),
            pl.BlockSpec((4, 1, 4, 256), lambda i: (0, 0, 0, 0)),
            pl.BlockSpec((1, 1), lambda i: (0, 0)),
        ],
        out_specs=pl.BlockSpec((Bt, 1), lambda i: (i, 0)),
        compiler_params=pltpu.CompilerParams(
            dimension_semantics=("parallel",)),
        cost_estimate=pl.CostEstimate(
            flops=2 * B * (256 * 64 * 64 + 64 * 1024 * 128 + 16 * 2048 * 256),
            transcendentals=B,
            bytes_accessed=B * 34 * 136 * 2 + B * 4),
    )(xin, perm, w1, conv1_b, conv2_w, conv2_b, conv3_w, conv3_b, fcw, fcb)
    return out
